# baseline (speedup 1.0000x reference)
import functools

import jax
import jax.numpy as jnp
from jax.experimental import pallas as pl
from jax.experimental.pallas import tpu as pltpu

_BN_EPS = 1e-5


def _stats_kernel(x_ref, acc_ref, *, valid_rows, blocks_per_core, need_mask):
    c = pl.program_id(0)
    j = pl.program_id(1)

    @pl.when(j == 0)
    def _():
        acc_ref[...] = jnp.zeros_like(acc_ref)

    blk = x_ref.shape[0]
    x = x_ref[...]
    if need_mask:
        gb = c * blocks_per_core + j
        rows = gb * blk + jax.lax.broadcasted_iota(jnp.int32, x.shape, 0)
        x = jnp.where(rows < valid_rows, x, 0.0)

    ones = jnp.ones((8, blk), jnp.float32)
    acc_ref[0:8, :] += jnp.dot(ones, x, preferred_element_type=jnp.float32)
    acc_ref[8:16, :] += jnp.dot(ones, x * x, preferred_element_type=jnp.float32)


def _apply_kernel(acc_ref, w_ref, g_ref, bt_ref, bi_ref, x_ref, o_ref,
                  w_s, v_s, *, inv_n):
    j = pl.program_id(1)

    @pl.when(j == 0)
    def _():
        tot = jnp.sum(acc_ref[...], axis=0)
        sums = tot[0:1, :]
        sqs = tot[8:9, :]
        mean = sums * inv_n
        var = jnp.maximum(sqs * inv_n - mean * mean, 0.0)
        s = g_ref[...] * jax.lax.rsqrt(var + _BN_EPS)
        c0 = bt_ref[...] - mean * s
        w_s[...] = w_ref[...] * s
        v_s[0:1, :] = jax.lax.dot_general(
            c0, w_ref[...], (((1,), (1,)), ((), ())),
            preferred_element_type=jnp.float32) + bi_ref[...]

    xb = x_ref[...]
    y = jax.lax.dot_general(xb, w_s[...], (((1,), (1,)), ((), ())),
                            preferred_element_type=jnp.float32)
    o_ref[...] = (y + v_s[0:1, :]).astype(o_ref.dtype)


def kernel(x, gamma, beta, weight, bias, *, block_rows=16384):
    n, d = x.shape

    if n >= 8:
        blk = min(max(8, (int(block_rows) // 8) * 8), (n // 8) * 8)
    else:
        blk = n
    grid_n = pl.cdiv(n, blk)
    n_split = 2 if grid_n >= 2 else 1
    gh = pl.cdiv(grid_n, n_split)
    ragged = (gh * n_split != grid_n) or (grid_n * blk != n)

    def blk_idx(c, j):
        g = c * gh + j
        return ((jnp.minimum(g, grid_n - 1), 0) if ragged else (g, 0))

    acc = pl.pallas_call(
        functools.partial(_stats_kernel, valid_rows=n,
                          blocks_per_core=gh, need_mask=ragged),
        out_shape=jax.ShapeDtypeStruct((n_split, 16, d), jnp.float32),
        grid=(n_split, gh),
        in_specs=[pl.BlockSpec((blk, d), blk_idx)],
        out_specs=pl.BlockSpec((None, 16, d), lambda c, j: (c, 0, 0)),
        compiler_params=pltpu.CompilerParams(
            dimension_semantics=("parallel", "arbitrary")),
    )(x)

    out = pl.pallas_call(
        functools.partial(_apply_kernel, inv_n=1.0 / n),
        out_shape=jax.ShapeDtypeStruct((n, d), x.dtype),
        grid=(n_split, gh),
        in_specs=[
            pl.BlockSpec((n_split, 16, d), lambda c, j: (0, 0, 0)),
            pl.BlockSpec((d, d), lambda c, j: (0, 0)),
            pl.BlockSpec((1, d), lambda c, j: (0, 0)),
            pl.BlockSpec((1, d), lambda c, j: (0, 0)),
            pl.BlockSpec((1, d), lambda c, j: (0, 0)),
            pl.BlockSpec((blk, d), blk_idx),
        ],
        out_specs=pl.BlockSpec((blk, d), blk_idx),
        scratch_shapes=[
            pltpu.VMEM((d, d), jnp.float32),
            pltpu.VMEM((8, d), jnp.float32),
        ],
        compiler_params=pltpu.CompilerParams(
            dimension_semantics=("parallel", "arbitrary")),
    )(acc, weight.astype(jnp.float32), gamma.reshape(1, d).astype(jnp.float32),
      beta.reshape(1, d).astype(jnp.float32), bias.reshape(1, d).astype(jnp.float32), x)

    return out
